# baseline (speedup 1.0000x reference)
import functools

import jax
import jax.numpy as jnp
import numpy as np
from jax import lax
from jax.experimental import pallas as pl
from jax.experimental.pallas import tpu as pltpu
from jax.experimental.pallas import tpu_sc as plsc

_B, _F0, _N, _E = 8, 128, 10000, 320000
_F1, _R = 128, 64
_NC, _NS = 2, 16
_CH = 160
_CG = _CH // 8
_EPT = _CH * 128
_EPAD = _NS * _EPT
_NP = 10240
_RPT = _NP // _NS
_ZR = 128

_mesh = plsc.VectorSubcoreMesh(core_axis_name="c", subcore_axis_name="s")

_PERM = np.array(
    [q * 32 + ofs + j
     for q in range(4) for j in range(16) for ofs in (0, 16)],
    dtype=np.int32,
).reshape(4, 16, 2).transpose(0, 1, 2).reshape(-1)


@functools.partial(
    pl.kernel,
    out_type=jax.ShapeDtypeStruct((_B, _NP, _F0), jnp.float32),
    mesh=_mesh,
    compiler_params=pltpu.CompilerParams(use_tc_tiling_on_sc=False, needs_layout_passes=False),
    scratch_types=[
        [pltpu.VMEM((24, 128), jnp.int32) for _ in range(2)],
        [pltpu.VMEM((64, 64), jnp.int32) for _ in range(2)],
        [pltpu.VMEM((64, 128), jnp.float32) for _ in range(2)],
        [pltpu.VMEM((1, 64), jnp.int32) for _ in range(2)],
        [pltpu.VMEM((1, 64), jnp.int32) for _ in range(2)],
        [pltpu.VMEM((1, 64), jnp.int32) for _ in range(2)],
        pltpu.VMEM((_ZR, 128), jnp.float32),
        pltpu.VMEM_SHARED((_NP, 128), jnp.float32),
        [pltpu.SemaphoreType.DMA for _ in range(2)],
        [pltpu.SemaphoreType.DMA for _ in range(2)],
        [pltpu.SemaphoreType.DMA for _ in range(2)],
    ],
)
def _gso(x_hbm, e_hbm, out_hbm,
         evs, gbufs, sbufs, idxbs, dstbs, dstsb, zbuf, acc,
         gsems, ssems, esems):
    cid = lax.axis_index("c")
    sid = lax.axis_index("s")
    z16 = jnp.zeros((16,), jnp.float32)

    def zrow(r, carry):
        for q in range(8):
            zbuf[r, pl.ds(q * 16, 16)] = z16
        return carry

    lax.fori_loop(0, _ZR, zrow, 0)

    def eload(j8, ge):
        pltpu.async_copy(e_hbm.at[sid, j8], evs[ge], esems[ge])

    def ewait(ge):
        pltpu.make_async_copy(e_hbm.at[sid, 0], evs[ge], esems[ge]).wait()

    def batch(i, carry0):
        b = cid * (_B // _NC) + i

        eload(0, 0)
        eload(1, 1)

        def zcopy(k2, c1):
            pltpu.sync_copy(zbuf, acc.at[pl.ds(sid * _RPT + k2 * _ZR, _ZR)])
            return c1

        lax.fori_loop(0, _RPT // _ZR, zcopy, 0)
        plsc.subcore_barrier()
        base = b * _NP

        def cgroup(j8, ge, ev):
            ewait(ge)

            def start_gather(k, p):
                row, half = k // 2, (k % 2) * 64
                ib = idxbs[p]
                db = dstbs[p]
                for q in range(4):
                    ib[0, pl.ds(q * 16, 16)] = (
                        ev[row, pl.ds(half + q * 16, 16)] + base)
                    db[0, pl.ds(q * 16, 16)] = (
                        ev[8 + row, pl.ds(half + q * 16, 16)])
                pltpu.async_copy(x_hbm.at[ib.at[0]], gbufs[p], gsems[p])

            def wait_gather(p):
                pltpu.make_async_copy(
                    x_hbm.at[idxbs[p].at[0]], gbufs[p], gsems[p]).wait()

            def wait_scatter(p):
                pltpu.make_async_copy(
                    sbufs[p], acc.at[dstsb[p].at[0]], ssems[p]).wait()

            start_gather(0, 0)
            start_gather(1, 1)

            def pair(jp, c2):
                for p in range(2):
                    k = jp * 2 + p

                    @pl.when(jp > 0)
                    def _():
                        wait_scatter(p)

                    wait_gather(p)
                    cur = gbufs[p]
                    sb = sbufs[p]

                    def scale(g, k=k, cur=cur, sb=sb):
                        row, half = k // 2, (k % 2) * 64
                        w16 = plsc.bitcast(
                            ev[16 + row, pl.ds(half + g * 16, 16)],
                            jnp.float32)
                        for l in range(16):
                            wv = w16[l]
                            e = g * 16 + l
                            for q in range(4):
                                ab = plsc.bitcast(
                                    cur[e, pl.ds(q * 16, 16)], jnp.bfloat16)
                                av, bv = plsc.unpack(
                                    ab, format=plsc.PackFormat.INTERLEAVED)
                                sb[e, pl.ds(q * 32, 16)] = av * wv
                                sb[e, pl.ds(q * 32 + 16, 16)] = bv * wv

                    plsc.parallel_loop(0, 4)(scale)
                    for q in range(4):
                        dstsb[p][0, pl.ds(q * 16, 16)] = (
                            dstbs[p][0, pl.ds(q * 16, 16)])
                    pltpu.async_copy(
                        sb, acc.at[dstsb[p].at[0]], ssems[p], add=True)

                    @pl.when(jp < 7)
                    def _():
                        start_gather(k + 2, p)
                return c2

            lax.fori_loop(0, 8, pair, 0)
            wait_scatter(0)
            wait_scatter(1)

            @pl.when(j8 + 2 < _CG)
            def _():
                eload(j8 + 2, ge)

        def gpair(jp2, c1):
            for ge in range(2):
                cgroup(jp2 * 2 + ge, ge, evs[ge])
            return c1

        lax.fori_loop(0, _CG // 2, gpair, 0)
        plsc.subcore_barrier()
        pltpu.sync_copy(acc.at[pl.ds(sid * _RPT, _RPT)],
                        out_hbm.at[b, pl.ds(sid * _RPT, _RPT)])
        plsc.subcore_barrier()
        return carry0

    lax.fori_loop(0, _B // _NC, batch, 0)


def _head(x0, x1, x2, w, bvec, wr, rb):

    nt = 1024

    def body(x0_ref, x1_ref, x2_ref, w_ref, b_ref, wr_ref, rb_ref, o_ref):
        z = jnp.dot(x0_ref[0], w_ref[0], preferred_element_type=jnp.float32)
        z = z + jnp.dot(x1_ref[0], w_ref[1], preferred_element_type=jnp.float32)
        z = z + jnp.dot(x2_ref[0], w_ref[2], preferred_element_type=jnp.float32)
        z = z + b_ref[0][None, :]
        y = jnp.maximum(z, 0.0)
        o = jnp.dot(y, wr_ref[...], preferred_element_type=jnp.float32)
        o_ref[0] = o + rb_ref[0][None, :]

    return pl.pallas_call(
        body,
        grid=(_B, _NP // nt),
        in_specs=[
            pl.BlockSpec((1, nt, _F0), lambda b, t: (b, t, 0)),
            pl.BlockSpec((1, nt, _F0), lambda b, t: (b, t, 0)),
            pl.BlockSpec((1, nt, _F0), lambda b, t: (b, t, 0)),
            pl.BlockSpec((3, _F0, _F1), lambda b, t: (0, 0, 0)),
            pl.BlockSpec((1, _F1), lambda b, t: (0, 0)),
            pl.BlockSpec((_F1, _R), lambda b, t: (0, 0)),
            pl.BlockSpec((1, _R), lambda b, t: (0, 0)),
        ],
        out_specs=pl.BlockSpec((1, nt, _R), lambda b, t: (b, t, 0)),
        out_shape=jax.ShapeDtypeStruct((_B, _NP, _R), jnp.float32),
    )(x0, x1, x2, w, bvec, wr, rb)


def kernel(x, edge_index, edge_weight, hconv_W, hconv_b, readout_W, readout_b):
    x0 = jnp.transpose(x, (0, 2, 1))
    x0 = jnp.pad(x0, ((0, 0), (0, _NP - _N), (0, 0)))
    pad = _EPAD - _E
    src = jnp.pad(edge_index[0], (0, pad)).reshape(_NS, _CG, 8, 128)
    dst = jnp.pad(edge_index[1], (0, pad)).reshape(_NS, _CG, 8, 128)
    w = jax.lax.bitcast_convert_type(
        jnp.pad(edge_weight, (0, pad)), jnp.int32
    ).reshape(_NS, _CG, 8, 128)
    edata = jnp.concatenate([src, dst, w], axis=2)
    def topack(a):
        ab = a[:, :, _PERM].astype(jnp.bfloat16)
        ab = jax.lax.bitcast_convert_type(
            ab.reshape(_B, _NP, _F0 // 2, 2), jnp.int32)
        return ab.reshape(_B * _NP, _F0 // 2)

    x1 = _gso(topack(x0), edata)
    x2 = _gso(topack(x1), edata)
    out = _head(x0, x1, x2, hconv_W, hconv_b.reshape(1, _F1),
                readout_W, readout_b.reshape(1, _R))
    return jnp.transpose(out[:, :_N, :], (0, 2, 1))


# speedup vs baseline: 1.1557x; 1.1557x over previous
import functools

import jax
import jax.numpy as jnp
import numpy as np
from jax import lax
from jax.experimental import pallas as pl
from jax.experimental.pallas import tpu as pltpu
from jax.experimental.pallas import tpu_sc as plsc

_B, _F0, _N, _E = 8, 128, 10000, 320000
_F1, _R = 128, 64
_NC, _NS = 2, 16
_CH = 160
_CG = _CH // 8
_EPT = _CH * 128
_EPAD = _NS * _EPT
_NP = 10240
_RPT = _NP // _NS
_ZR = 128

_mesh = plsc.VectorSubcoreMesh(core_axis_name="c", subcore_axis_name="s")

_PERM = np.array(
    [q * 32 + ofs + j
     for q in range(4) for j in range(16) for ofs in (0, 16)],
    dtype=np.int32,
).reshape(4, 16, 2).transpose(0, 1, 2).reshape(-1)


@functools.partial(
    pl.kernel,
    out_type=jax.ShapeDtypeStruct((_B, _NP, _F0), jnp.float32),
    mesh=_mesh,
    compiler_params=pltpu.CompilerParams(use_tc_tiling_on_sc=False, needs_layout_passes=False),
    scratch_types=[
        pltpu.VMEM((24, 128), jnp.int32),
        [pltpu.VMEM((64, 64), jnp.int32) for _ in range(2)],
        [pltpu.VMEM((64, 128), jnp.float32) for _ in range(2)],
        [pltpu.VMEM((1, 64), jnp.int32) for _ in range(2)],
        [pltpu.VMEM((1, 64), jnp.int32) for _ in range(2)],
        [pltpu.VMEM((1, 64), jnp.int32) for _ in range(2)],
        pltpu.VMEM((_ZR, 128), jnp.float32),
        pltpu.VMEM_SHARED((_NP, 128), jnp.float32),
        [pltpu.SemaphoreType.DMA for _ in range(2)],
        [pltpu.SemaphoreType.DMA for _ in range(2)],
    ],
)
def _gso(x_hbm, e_hbm, out_hbm,
         ev, gbufs, sbufs, idxbs, dstbs, dstsb, zbuf, acc,
         gsems, ssems):
    cid = lax.axis_index("c")
    sid = lax.axis_index("s")
    z16 = jnp.zeros((16,), jnp.float32)

    def zrow(r, carry):
        for q in range(8):
            zbuf[r, pl.ds(q * 16, 16)] = z16
        return carry

    lax.fori_loop(0, _ZR, zrow, 0)

    def batch(i, carry0):
        b = cid * (_B // _NC) + i

        def zcopy(k2, c1):
            pltpu.sync_copy(zbuf, acc.at[pl.ds(sid * _RPT + k2 * _ZR, _ZR)])
            return c1

        lax.fori_loop(0, _RPT // _ZR, zcopy, 0)
        plsc.subcore_barrier()
        base = b * _NP

        def cgroup(j8, c1):
            pltpu.sync_copy(e_hbm.at[sid, j8], ev)

            def start_gather(k, p):
                row, half = k // 2, (k % 2) * 64
                ib = idxbs[p]
                db = dstbs[p]
                for q in range(4):
                    ib[0, pl.ds(q * 16, 16)] = (
                        ev[row, pl.ds(half + q * 16, 16)] + base)
                    db[0, pl.ds(q * 16, 16)] = (
                        ev[8 + row, pl.ds(half + q * 16, 16)])
                pltpu.async_copy(x_hbm.at[ib.at[0]], gbufs[p], gsems[p])

            def wait_gather(p):
                pltpu.make_async_copy(
                    x_hbm.at[idxbs[p].at[0]], gbufs[p], gsems[p]).wait()

            def wait_scatter(p):
                pltpu.make_async_copy(
                    sbufs[p], acc.at[dstsb[p].at[0]], ssems[p]).wait()

            start_gather(0, 0)
            start_gather(1, 1)

            def pair(jp, c2):
                for p in range(2):
                    k = jp * 2 + p

                    @pl.when(jp > 0)
                    def _():
                        wait_scatter(p)

                    wait_gather(p)
                    cur = gbufs[p]
                    sb = sbufs[p]

                    def scale(g, k=k, cur=cur, sb=sb):
                        row, half = k // 2, (k % 2) * 64
                        w16 = plsc.bitcast(
                            ev[16 + row, pl.ds(half + g * 16, 16)],
                            jnp.float32)
                        for l in range(16):
                            wv = w16[l]
                            e = g * 16 + l
                            for q in range(4):
                                ab = plsc.bitcast(
                                    cur[e, pl.ds(q * 16, 16)], jnp.bfloat16)
                                av, bv = plsc.unpack(
                                    ab, format=plsc.PackFormat.INTERLEAVED)
                                sb[e, pl.ds(q * 32, 16)] = av * wv
                                sb[e, pl.ds(q * 32 + 16, 16)] = bv * wv

                    plsc.parallel_loop(0, 4)(scale)
                    for q in range(4):
                        dstsb[p][0, pl.ds(q * 16, 16)] = (
                            dstbs[p][0, pl.ds(q * 16, 16)])
                    pltpu.async_copy(
                        sb, acc.at[dstsb[p].at[0]], ssems[p], add=True)

                    @pl.when(jp < 7)
                    def _():
                        start_gather(k + 2, p)
                return c2

            lax.fori_loop(0, 8, pair, 0)
            wait_scatter(0)
            wait_scatter(1)
            return c1

        lax.fori_loop(0, _CG, cgroup, 0)
        plsc.subcore_barrier()
        pltpu.sync_copy(acc.at[pl.ds(sid * _RPT, _RPT)],
                        out_hbm.at[b, pl.ds(sid * _RPT, _RPT)])
        plsc.subcore_barrier()
        return carry0

    lax.fori_loop(0, _B // _NC, batch, 0)


def _head(x0, x1, x2, w, bvec, wr, rb):

    nt = 1024

    def body(x0_ref, x1_ref, x2_ref, w_ref, b_ref, wr_ref, rb_ref, o_ref):
        z = jnp.dot(x0_ref[0], w_ref[0], preferred_element_type=jnp.float32)
        z = z + jnp.dot(x1_ref[0], w_ref[1], preferred_element_type=jnp.float32)
        z = z + jnp.dot(x2_ref[0], w_ref[2], preferred_element_type=jnp.float32)
        z = z + b_ref[0][None, :]
        y = jnp.maximum(z, 0.0)
        o = jnp.dot(y, wr_ref[...], preferred_element_type=jnp.float32)
        o_ref[0] = o + rb_ref[0][None, :]

    return pl.pallas_call(
        body,
        grid=(_B, _NP // nt),
        in_specs=[
            pl.BlockSpec((1, nt, _F0), lambda b, t: (b, t, 0)),
            pl.BlockSpec((1, nt, _F0), lambda b, t: (b, t, 0)),
            pl.BlockSpec((1, nt, _F0), lambda b, t: (b, t, 0)),
            pl.BlockSpec((3, _F0, _F1), lambda b, t: (0, 0, 0)),
            pl.BlockSpec((1, _F1), lambda b, t: (0, 0)),
            pl.BlockSpec((_F1, _R), lambda b, t: (0, 0)),
            pl.BlockSpec((1, _R), lambda b, t: (0, 0)),
        ],
        out_specs=pl.BlockSpec((1, nt, _R), lambda b, t: (b, t, 0)),
        out_shape=jax.ShapeDtypeStruct((_B, _NP, _R), jnp.float32),
    )(x0, x1, x2, w, bvec, wr, rb)


def kernel(x, edge_index, edge_weight, hconv_W, hconv_b, readout_W, readout_b):
    x0 = jnp.transpose(x, (0, 2, 1))
    x0 = jnp.pad(x0, ((0, 0), (0, _NP - _N), (0, 0)))
    pad = _EPAD - _E
    src = jnp.pad(edge_index[0], (0, pad)).reshape(_NS, _CG, 8, 128)
    dst = jnp.pad(edge_index[1], (0, pad)).reshape(_NS, _CG, 8, 128)
    w = jax.lax.bitcast_convert_type(
        jnp.pad(edge_weight, (0, pad)), jnp.int32
    ).reshape(_NS, _CG, 8, 128)
    edata = jnp.concatenate([src, dst, w], axis=2)
    def topack(a):
        ab = a[:, :, _PERM].astype(jnp.bfloat16)
        ab = jax.lax.bitcast_convert_type(
            ab.reshape(_B, _NP, _F0 // 2, 2), jnp.int32)
        return ab.reshape(_B * _NP, _F0 // 2)

    x1 = _gso(topack(x0), edata)
    x2 = _gso(topack(x1), edata)
    out = _head(x0, x1, x2, hconv_W, hconv_b.reshape(1, _F1),
                readout_W, readout_b.reshape(1, _R))
    return jnp.transpose(out[:, :_N, :], (0, 2, 1))
